# baseline (speedup 1.0000x reference)
import functools

import jax
import jax.numpy as jnp
from jax import lax
from jax.experimental import pallas as pl
from jax.experimental.pallas import tpu as pltpu
from jax.experimental.pallas import tpu_sc as plsc

N = 4096
M = 4096

SC_ROWS = 4096
W = 32
RPW = SC_ROWS // W
GRP = RPW // 8
HALF = GRP // 2

TC_ROWS = N - SC_ROWS
BM = 512
TC_GRID = TC_ROWS // BM if TC_ROWS else 0


GROUPS = RPW // 16
HCOL = N // 2


def _sc_body(A_ref, Q_ref, AT_ref, x_ref, y_ref,
             out_ref,
             x_loc, y_loc, stage,
             buf0, buf1, sem0, sem1):
    wid = lax.axis_index("s") * 2 + lax.axis_index("c")
    row0 = wid * RPW

    pltpu.sync_copy(x_ref, x_loc)
    pltpu.sync_copy(y_ref, y_loc)

    zeros16 = (jnp.zeros((16,), jnp.float32),) * 16
    nhalf = GROUPS * 2

    def start_dma(mat_ref, idx, buf, sem):
        g, h = idx // 2, idx % 2
        pltpu.async_copy(
            mat_ref.at[pl.ds(row0 + g * 16, 16), pl.ds(h * HCOL, HCOL)],
            buf, sem)

    def phase(m_idx, mat_ref, vec_loc):
        start_dma(mat_ref, 0, buf0, sem0)
        start_dma(mat_ref, 1, buf1, sem1)
        accs = zeros16
        for idx in range(nhalf):
            buf = buf0 if idx % 2 == 0 else buf1
            sem = sem0 if idx % 2 == 0 else sem1
            h = idx % 2
            pltpu.make_async_copy(
                mat_ref.at[pl.ds(0, 16), pl.ds(0, HCOL)], buf, sem).wait()

            def ch(j4, accs, buf=buf, vec_loc=vec_loc, h=h):
                for dj in range(4):
                    j = j4 * 4 + dj
                    xc = vec_loc[pl.ds(h * HCOL + j * 16, 16)]
                    accs = tuple(
                        accs[r] + buf[r, pl.ds(j * 16, 16)] * xc
                        for r in range(16))
                return accs

            accs = lax.fori_loop(0, HCOL // 64, ch, accs)
            if idx + 2 < nhalf:
                start_dma(mat_ref, idx + 2, buf, sem)
            if h == 1:
                g = idx // 2
                for r in range(16):
                    stage[pl.ds((g * 16 + r) * 16, 16)] = accs[r]
                accs = zeros16
        pltpu.sync_copy(
            stage, out_ref.at[pl.ds((m_idx * SC_ROWS + row0) * 16, RPW * 16)])

    phase(0, A_ref, x_loc)
    phase(1, Q_ref, x_loc)
    phase(2, AT_ref, y_loc)


def _sc_partials(A, Q, AT, x, y):
    mesh = plsc.VectorSubcoreMesh(core_axis_name="c", subcore_axis_name="s")
    f = functools.partial(
        pl.kernel,
        mesh=mesh,
        out_type=jax.ShapeDtypeStruct((3 * SC_ROWS * 16,), jnp.float32),
        scratch_types=[
            pltpu.VMEM((N,), jnp.float32),
            pltpu.VMEM((M,), jnp.float32),
            pltpu.VMEM((RPW * 16,), jnp.float32),
            pltpu.VMEM((16, HCOL), jnp.float32),
            pltpu.VMEM((16, HCOL), jnp.float32),
            pltpu.SemaphoreType.DMA,
            pltpu.SemaphoreType.DMA,
        ],
    )(_sc_body)
    return f(A, Q, AT, x, y)


def _tc_body(x_ref, y_ref, b_ref, c_ref, iy_ref, xb_ref, yb_ref,
             Q_ref, A_ref, AT_ref,
             s1_ref, s2_ref, squad_ref, acc_ref):
    i = pl.program_id(0)
    xT = x_ref[...]
    yT = y_ref[...]
    b_blk = b_ref[...]
    c_blk = c_ref[...]
    iy_blk = iy_ref[...]
    x_blk = xb_ref[...]

    Ax = jnp.sum(A_ref[...] * xT, axis=1, keepdims=True)
    part1 = Ax - b_blk
    part1 = part1 + iy_blk * jnp.maximum(-part1, 0.0)
    s1 = jnp.sum(part1 * part1)

    Qx = jnp.sum(Q_ref[...] * xT, axis=1, keepdims=True)
    ATy = jnp.sum(AT_ref[...] * yT, axis=1, keepdims=True)
    d = Qx + ATy + c_blk
    s2 = jnp.sum(d * d)
    squad = jnp.sum(x_blk * Qx)

    parts = (s1, s2, squad)

    @pl.when(i == 0)
    def _init():
        for k, v in enumerate(parts):
            acc_ref[k] = v

    @pl.when(i != 0)
    def _accum():
        for k, v in enumerate(parts):
            acc_ref[k] = acc_ref[k] + v

    @pl.when(i == TC_GRID - 1)
    def _fini():
        s1_ref[0, 0] = acc_ref[0]
        s2_ref[0, 0] = acc_ref[1]
        squad_ref[0, 0] = acc_ref[2]


def _tc_partials(Q, A, AT, xT, yT, b2, c2, iy2, x, y):
    blk0 = SC_ROWS // BM
    full_vec = pl.BlockSpec((1, N), lambda i: (0, 0))
    blk_vec = pl.BlockSpec((BM, 1), lambda i: (i + blk0, 0))
    row_blk = pl.BlockSpec((BM, N), lambda i: (i + blk0, 0))
    scalar_out = pl.BlockSpec((1, 1), lambda i: (0, 0),
                              memory_space=pltpu.SMEM)
    return pl.pallas_call(
        _tc_body,
        grid=(TC_GRID,),
        in_specs=[full_vec, full_vec, blk_vec, blk_vec, blk_vec, blk_vec,
                  blk_vec, row_blk, row_blk, row_blk],
        out_specs=[scalar_out] * 3,
        out_shape=[jax.ShapeDtypeStruct((1, 1), jnp.float32)] * 3,
        scratch_shapes=[pltpu.SMEM((3,), jnp.float32)],
    )(xT, yT, b2, c2, iy2, x, y, Q, A, AT)


def _combine_body(x_ref, y_ref, b_ref, c_ref, iy_ref, sc_ref,
                  s1_ref, s2_ref, squad_ref,
                  res_ref, t1_ref, t2_ref, t3_ref):
    x_col = x_ref[...]
    y_col = y_ref[...]
    b_col = b_ref[...]
    c_col = c_ref[...]
    iy_col = iy_ref[...]

    dots = jnp.sum(sc_ref[...], axis=1, keepdims=True)
    Ad = dots[0:SC_ROWS]
    Qd = dots[SC_ROWS:2 * SC_ROWS]
    ATd = dots[2 * SC_ROWS:3 * SC_ROWS]

    bs = b_col[0:SC_ROWS]
    part1 = Ad - bs
    part1 = part1 + iy_col[0:SC_ROWS] * jnp.maximum(-part1, 0.0)
    s1 = jnp.sum(part1 * part1) + s1_ref[0, 0]

    d = Qd + ATd + c_col[0:SC_ROWS]
    s2 = jnp.sum(d * d) + s2_ref[0, 0]
    squad = jnp.sum(x_col[0:SC_ROWS] * Qd) + squad_ref[0, 0]

    slin = jnp.sum(c_col * x_col)
    svio = jnp.sum(b_col * y_col)
    sb2 = jnp.sum(b_col * b_col)
    sc2 = jnp.sum(c_col * c_col)

    t1 = jnp.sqrt(s1) / (0.0001 + jnp.sqrt(sb2))
    t2 = jnp.sqrt(s2) / (0.0001 + jnp.sqrt(sc2))
    t3 = jnp.abs(squad + slin + svio)
    t1_ref[0, 0] = t1
    t2_ref[0, 0] = t2
    t3_ref[0, 0] = t3
    res_ref[0, 0] = t1 + t2 + t3


def _combine(x, y, b2, c2, iy2, sc_out, s1t, s2t, squadt):
    scalar_out = pl.BlockSpec(memory_space=pltpu.SMEM)
    scalar_in = pl.BlockSpec(memory_space=pltpu.SMEM)
    return pl.pallas_call(
        _combine_body,
        in_specs=[pl.BlockSpec()] * 6 + [scalar_in] * 3,
        out_specs=[scalar_out] * 4,
        out_shape=[jax.ShapeDtypeStruct((1, 1), jnp.float32)] * 4,
    )(x, y, b2, c2, iy2, sc_out, s1t, s2t, squadt)


def kernel(Q, A, AT, b, c, x, y, Iy, il, iu, l, u):
    xf = x[:, 0]
    yf = y[:, 0]
    xT = x.T
    yT = y.T

    if TC_ROWS:
        s1t, s2t, squadt = _tc_partials(
            Q, A, AT, xT, yT, b[:, None], c[:, None], Iy[:, None], x, y)
    else:
        z = jnp.zeros((1, 1), jnp.float32)
        s1t, s2t, squadt = z, z, z

    sc_out = _sc_partials(A, Q, AT, xf, yf).reshape(3 * SC_ROWS, 16)

    res, t1, t2, t3 = _combine(
        x, y, b[:, None], c[:, None], Iy[:, None], sc_out, s1t, s2t, squadt)
    return (res, t1[0, 0], t2[0, 0], t3)


# speedup vs baseline: 1.8013x; 1.8013x over previous
import jax
import jax.numpy as jnp
from jax.experimental import pallas as pl
from jax.experimental.pallas import tpu as pltpu

N = 4096
M = 4096
BM = 256
GRID = M // BM


def _body(x_ref, y_ref, b_ref, c_ref, iy_ref, xb_ref, yb_ref,
          Q_ref, A_ref, AT_ref,
          res_ref, t1_ref, t2_ref, t3_ref, acc_ref):
    i = pl.program_id(0)

    xT = x_ref[...]
    yT = y_ref[...]
    b_blk = b_ref[...]
    c_blk = c_ref[...]
    iy_blk = iy_ref[...]
    x_blk = xb_ref[...]
    y_blk = yb_ref[...]

    Ax = jnp.sum(A_ref[...] * xT, axis=1, keepdims=True)
    part1 = Ax - b_blk
    part1 = part1 + iy_blk * jnp.maximum(-part1, 0.0)
    s1 = jnp.sum(part1 * part1)

    Qx = jnp.sum(Q_ref[...] * xT, axis=1, keepdims=True)
    ATy = jnp.sum(AT_ref[...] * yT, axis=1, keepdims=True)
    d = Qx + ATy + c_blk
    s2 = jnp.sum(d * d)

    squad = jnp.sum(x_blk * Qx)
    slin = jnp.sum(c_blk * x_blk)
    svio = jnp.sum(b_blk * y_blk)
    sb2 = jnp.sum(b_blk * b_blk)
    sc2 = jnp.sum(c_blk * c_blk)

    parts = (s1, s2, squad, slin, svio, sb2, sc2)

    @pl.when(i == 0)
    def _init():
        for k, v in enumerate(parts):
            acc_ref[k] = v

    @pl.when(i != 0)
    def _accum():
        for k, v in enumerate(parts):
            acc_ref[k] = acc_ref[k] + v

    @pl.when(i == GRID - 1)
    def _fini():
        t1 = jnp.sqrt(acc_ref[0]) / (0.0001 + jnp.sqrt(acc_ref[5]))
        t2 = jnp.sqrt(acc_ref[1]) / (0.0001 + jnp.sqrt(acc_ref[6]))
        t3 = jnp.abs(acc_ref[2] + acc_ref[3] + acc_ref[4])
        t1_ref[0, 0] = t1
        t2_ref[0, 0] = t2
        t3_ref[0, 0] = t3
        res_ref[0, 0] = t1 + t2 + t3


def kernel(Q, A, AT, b, c, x, y, Iy, il, iu, l, u):
    b2 = b[:, None]
    c2 = c[:, None]
    iy2 = Iy[:, None]
    xT = x.T
    yT = y.T

    out_shapes = [jax.ShapeDtypeStruct((1, 1), jnp.float32)] * 4
    full_vec = pl.BlockSpec((1, N), lambda i: (0, 0))
    blk_vec = pl.BlockSpec((BM, 1), lambda i: (i, 0))
    row_blk = pl.BlockSpec((BM, N), lambda i: (i, 0))
    scalar_out = pl.BlockSpec((1, 1), lambda i: (0, 0), memory_space=pltpu.SMEM)

    res, t1, t2, t3 = pl.pallas_call(
        _body,
        grid=(GRID,),
        in_specs=[full_vec, full_vec, blk_vec, blk_vec, blk_vec, blk_vec,
                  blk_vec, row_blk, row_blk, row_blk],
        out_specs=[scalar_out] * 4,
        out_shape=out_shapes,
        scratch_shapes=[pltpu.SMEM((7,), jnp.float32)],
    )(xT, yT, b2, c2, iy2, x, y, Q, A, AT)

    return (res, t1[0, 0], t2[0, 0], t3)
